# baseline (speedup 1.0000x reference)
import jax
import jax.numpy as jnp
from jax import lax
from jax.experimental import pallas as pl
from jax.experimental.pallas import tpu as pltpu
from jax.experimental.pallas import tpu_sc as plsc

N = 10000
D = 128
E = 320000

NC = 2
NS = 16
NW = NC * NS
L = 16

C = 120
NR = 3
NI = 4
UNROLL = 12
GT = 168
G0 = 96
G1 = GT - G0
EPAD = NS * GT * C
CW = 1008
GW = EPAD // (NW * CW)
NPAD = 10112
RPW = NPAD // NS


def _proj_body(h_ref, wfc_ref, p_ref, z_ref, s2_ref):
    z = lax.dot_general(h_ref[...], wfc_ref[...], (((1,), (1,)), ((), ())),
                        preferred_element_type=jnp.float32)
    z_ref[...] = z
    s2_ref[...] = lax.dot_general(z, p_ref[...], (((1,), (0,)), ((), ())),
                                  preferred_element_type=jnp.float32)


def _combine_body(acc_ref, d_ref, out_ref):
    a = acc_ref[0] + acc_ref[1]
    dsum = d_ref[0] + d_ref[1]
    safe = jnp.where(dsum > 0, dsum, 1.0)
    out_ref[...] = jnp.where(dsum > 0, a / safe, 0.0)


def _w_body(sl_hbm, sr_hbm, src_hbm, dst_hbm, w_hbm,
            sl_v, sr_v, ss0, ss1, sd0, sd1, wo0, wo1,
            smi0, smi1, smo0, smo1):
    cid = lax.axis_index("c")
    sid = lax.axis_index("s")
    wid = sid * NC + cid
    row0 = wid * GW
    ss = [ss0, ss1]
    sd = [sd0, sd1]
    wo = [wo0, wo1]
    sem_i = [smi0, smi1]
    sem_o = [smo0, smo1]

    pltpu.sync_copy(sl_hbm, sl_v)
    pltpu.sync_copy(sr_hbm, sr_v)
    pltpu.async_copy(src_hbm.at[row0], ss[0], sem_i[0])
    pltpu.async_copy(dst_hbm.at[row0], sd[0], sem_i[0])

    def _pair(t, _):
        for b in range(2):
            g = 2 * t + b

            @pl.when(g + 1 < GW)
            def _():
                pltpu.async_copy(src_hbm.at[row0 + g + 1], ss[1 - b],
                                 sem_i[1 - b])
                pltpu.async_copy(dst_hbm.at[row0 + g + 1], sd[1 - b],
                                 sem_i[1 - b])
            pltpu.make_async_copy(src_hbm.at[row0 + g], ss[b],
                                  sem_i[b]).wait()
            pltpu.make_async_copy(dst_hbm.at[row0 + g], sd[b],
                                  sem_i[b]).wait()

            @pl.when(g >= 2)
            def _():
                pltpu.make_async_copy(wo[b], w_hbm.at[row0 + g - 2],
                                      sem_o[b]).wait()
            for j in range(CW // L):
                s16 = ss[b][pl.ds(j * L, L)]
                d16 = jnp.minimum(sd[b][pl.ds(j * L, L)], N - 1)
                e16 = (plsc.load_gather(sl_v, [s16])
                       + plsc.load_gather(sr_v, [d16]))
                e16 = jnp.where(e16 >= 0, e16, e16 * jnp.float32(0.01))
                wo[b][pl.ds(j * L, L)] = jnp.exp(e16)
            pltpu.async_copy(wo[b], w_hbm.at[row0 + g], sem_o[b])
        return 0

    lax.fori_loop(0, GW // 2, _pair, 0)
    for g in range(GW - 2, GW):
        pltpu.make_async_copy(wo[g % 2], w_hbm.at[row0 + g],
                              sem_o[g % 2]).wait()


def _agg_body(z_hbm, src_hbm, dst_hbm, w_hbm, acc_hbm, d_hbm,
              rb0, rb1, rb2, is0, is1, is2, is3, id0, id1, id2, id3,
              wv0, wv1, wv2, wv3, dzero_v, acc_sh, d_sh,
              sg0, sg1, sg2, ss0, ss1, ss2,
              smi0, smi1, smi2, smi3, smw0, smw1, smw2, smw3, sem_d):
    cid = lax.axis_index("c")
    sid = lax.axis_index("s")
    rows = [rb0, rb1, rb2]
    isv = [is0, is1, is2, is3]
    idv = [id0, id1, id2, id3]
    wv = [wv0, wv1, wv2, wv3]
    sem_g = [sg0, sg1, sg2]
    sem_s = [ss0, ss1, ss2]
    sem_i = [smi0, smi1, smi2, smi3]
    sem_w = [smw0, smw1, smw2, smw3]
    zero16 = jnp.zeros((L,), jnp.float32)

    row0 = jnp.where(cid == 0, sid * G0, NS * G0 + sid * G1)
    gw = jnp.where(cid == 0, G0, G1)

    def _zrow(r, _):
        for k in range(D // L):
            rb0[r, pl.ds(k * L, L)] = zero16
        return 0
    lax.fori_loop(0, C, _zrow, 0)

    def _zd(i, _):
        dzero_v[pl.ds(i * L, L)] = zero16
        return 0
    lax.fori_loop(0, 640 // L, _zd, 0)

    r0 = sid * RPW
    for j in range(RPW // C):
        pltpu.sync_copy(rb0, acc_sh.at[pl.ds(r0 + j * C, C)])
    rem = RPW % C
    if rem:
        pltpu.sync_copy(rb0.at[pl.ds(0, rem)],
                        acc_sh.at[pl.ds(r0 + (RPW // C) * C, rem)])
    pltpu.sync_copy(dzero_v.at[pl.ds(0, RPW)], d_sh.at[pl.ds(r0, RPW)])

    for p in range(2):
        pltpu.async_copy(src_hbm.at[row0 + p], isv[p], sem_i[p])
        pltpu.async_copy(dst_hbm.at[row0 + p], idv[p], sem_i[p])
        pltpu.async_copy(w_hbm.at[row0 + p], wv[p], sem_w[p])
    pltpu.make_async_copy(src_hbm.at[row0], isv[0], sem_i[0]).wait()
    pltpu.make_async_copy(dst_hbm.at[row0], idv[0], sem_i[0]).wait()
    pltpu.async_copy(z_hbm.at[isv[0]], rows[0], sem_g[0])
    plsc.subcore_barrier()

    def _block(t, _):
        for b in range(UNROLL):
            g = t * UNROLL + b
            br = b % NR
            brn = (b + 1) % NR
            bi = b % NI
            bin_ = (b + 1) % NI
            bi2p = (b + 2) % NI

            @pl.when(g >= 2)
            def _():
                pltpu.make_async_copy(
                    rows[brn], acc_sh.at[idv[bi2p]], sem_s[brn]).wait()
                pltpu.make_async_copy(
                    wv[bi2p], d_sh.at[idv[bi2p]], sem_d).wait()
            @pl.when(g + 2 < gw)
            def _():
                pltpu.async_copy(src_hbm.at[row0 + g + 2], isv[bi2p],
                                 sem_i[bi2p])
                pltpu.async_copy(dst_hbm.at[row0 + g + 2], idv[bi2p],
                                 sem_i[bi2p])
                pltpu.async_copy(w_hbm.at[row0 + g + 2], wv[bi2p],
                                 sem_w[bi2p])
            @pl.when(g + 1 < gw)
            def _():
                pltpu.make_async_copy(src_hbm.at[row0 + g + 1], isv[bin_],
                                      sem_i[bin_]).wait()
                pltpu.make_async_copy(dst_hbm.at[row0 + g + 1], idv[bin_],
                                      sem_i[bin_]).wait()
                pltpu.async_copy(z_hbm.at[isv[bin_]], rows[brn], sem_g[brn])
            pltpu.make_async_copy(z_hbm.at[isv[bi]], rows[br],
                                  sem_g[br]).wait()
            pltpu.make_async_copy(w_hbm.at[row0 + g], wv[bi],
                                  sem_w[bi]).wait()

            @plsc.parallel_loop(0, C, unroll=2)
            def _scale(r):
                wb = plsc.load_gather(wv[bi], [jnp.full((L,), r, jnp.int32)])
                for k in range(D // L):
                    sl_ = pl.ds(k * L, L)
                    rows[br][r, sl_] = rows[br][r, sl_] * wb

            pltpu.async_copy(rows[br], acc_sh.at[idv[bi]], sem_s[br],
                             add=True)
            pltpu.async_copy(wv[bi], d_sh.at[idv[bi]], sem_d, add=True)
        return 0

    lax.fori_loop(0, gw // UNROLL, _block, 0)
    for goff in (2, 1):
        br = (-goff) % NR
        bi = (-goff) % NI
        pltpu.make_async_copy(rows[br], acc_sh.at[idv[bi]],
                              sem_s[br]).wait()
        pltpu.make_async_copy(wv[bi], d_sh.at[idv[bi]], sem_d).wait()
    plsc.subcore_barrier()

    pltpu.sync_copy(acc_sh.at[pl.ds(r0, RPW)], acc_hbm.at[cid, pl.ds(r0, RPW)])
    pltpu.sync_copy(d_sh.at[pl.ds(r0, RPW)], dzero_v.at[pl.ds(0, RPW)])
    pltpu.sync_copy(dzero_v.at[pl.ds(0, RPW)],
                    d_hbm.at[pl.ds(cid * NPAD + r0, RPW)])


def kernel(h, edge_index, W_fc, W_attn):
    src = edge_index[0].astype(jnp.int32)
    dst = edge_index[1].astype(jnp.int32)
    pad = EPAD - E
    src_p = jnp.pad(src, (0, pad)).reshape(NS * GT, C)
    dst_p = jnp.pad(dst, (0, pad), constant_values=N).reshape(NS * GT, C)

    a_l = W_attn[0, :D]
    a_r = W_attn[0, D:]
    P = jnp.zeros((D, D), jnp.float32).at[:, 0].set(a_l).at[:, 1].set(a_r)

    blk = 1000
    z, s2 = pl.pallas_call(
        _proj_body,
        grid=(N // blk,),
        in_specs=[
            pl.BlockSpec((blk, D), lambda i: (i, 0)),
            pl.BlockSpec((D, D), lambda i: (0, 0)),
            pl.BlockSpec((D, D), lambda i: (0, 0)),
        ],
        out_specs=[
            pl.BlockSpec((blk, D), lambda i: (i, 0)),
            pl.BlockSpec((blk, D), lambda i: (i, 0)),
        ],
        out_shape=[
            jax.ShapeDtypeStruct((N, D), jnp.float32),
            jax.ShapeDtypeStruct((N, D), jnp.float32),
        ],
    )(h, W_fc, P)
    sl = s2[:, 0]
    sr = s2[:, 1]

    mesh = plsc.VectorSubcoreMesh(core_axis_name="c", subcore_axis_name="s")
    scp = pltpu.CompilerParams(needs_layout_passes=False)

    w = pl.kernel(
        _w_body,
        out_type=jax.ShapeDtypeStruct((NW * GW, CW), jnp.float32),
        mesh=mesh,
        compiler_params=scp,
        scratch_types=(
            [
                pltpu.VMEM((N,), jnp.float32),
                pltpu.VMEM((N,), jnp.float32),
            ]
            + [pltpu.VMEM((CW,), jnp.int32) for _ in range(4)]
            + [pltpu.VMEM((CW,), jnp.float32) for _ in range(2)]
            + [pltpu.SemaphoreType.DMA for _ in range(4)]
        ),
    )(sl, sr, src_p.reshape(NW * GW, CW), dst_p.reshape(NW * GW, CW))

    acc2, d2 = pl.kernel(
        _agg_body,
        out_type=[
            jax.ShapeDtypeStruct((NC, NPAD, D), jnp.float32),
            jax.ShapeDtypeStruct((NC * NPAD,), jnp.float32),
        ],
        mesh=mesh,
        compiler_params=scp,
        scratch_types=(
            [pltpu.VMEM((C, D), jnp.float32) for _ in range(NR)]
            + [pltpu.VMEM((C,), jnp.int32) for _ in range(2 * NI)]
            + [pltpu.VMEM((C,), jnp.float32) for _ in range(NI)]
            + [
                pltpu.VMEM((640,), jnp.float32),
                pltpu.MemorySpace.VMEM_SHARED((NPAD, D), jnp.float32),
                pltpu.MemorySpace.VMEM_SHARED((NPAD,), jnp.float32),
            ]
            + [pltpu.SemaphoreType.DMA for _ in range(15)]
        ),
    )(z, src_p, dst_p, w.reshape(NS * GT, C))

    out = pl.pallas_call(
        _combine_body,
        grid=(N // blk,),
        in_specs=[
            pl.BlockSpec((NC, blk, D), lambda i: (0, i, 0)),
            pl.BlockSpec((NC, blk, 1), lambda i: (0, i, 0)),
        ],
        out_specs=pl.BlockSpec((blk, D), lambda i: (i, 0)),
        out_shape=jax.ShapeDtypeStruct((N, D), jnp.float32),
    )(acc2, d2.reshape(NC, NPAD, 1))
    return out


# speedup vs baseline: 1.0673x; 1.0673x over previous
import jax
import jax.numpy as jnp
from jax import lax
from jax.experimental import pallas as pl
from jax.experimental.pallas import tpu as pltpu
from jax.experimental.pallas import tpu_sc as plsc

N = 10000
D = 128
E = 320000

NC = 2
NS = 16
NW = NC * NS
L = 16

C = 120
NR = 3
NI = 4
UNROLL = 12
GT = 168
G0 = 120
G1 = GT - G0
EPAD = NS * GT * C
CW = 1008
GW = EPAD // (NW * CW)
NPAD = 10112
RPW = NPAD // NS


def _proj_body(h_ref, wfc_ref, p_ref, z_ref, s2_ref):
    z = lax.dot_general(h_ref[...], wfc_ref[...], (((1,), (1,)), ((), ())),
                        preferred_element_type=jnp.float32)
    z_ref[...] = z
    s2_ref[...] = lax.dot_general(z, p_ref[...], (((1,), (0,)), ((), ())),
                                  preferred_element_type=jnp.float32)


def _combine_body(acc_ref, d_ref, out_ref):
    a = acc_ref[0] + acc_ref[1]
    dsum = d_ref[0] + d_ref[1]
    safe = jnp.where(dsum > 0, dsum, 1.0)
    out_ref[...] = jnp.where(dsum > 0, a / safe, 0.0)


def _w_body(sl_hbm, sr_hbm, src_hbm, dst_hbm, w_hbm,
            sl_v, sr_v, ss0, ss1, sd0, sd1, wo0, wo1,
            smi0, smi1, smo0, smo1):
    cid = lax.axis_index("c")
    sid = lax.axis_index("s")
    wid = sid * NC + cid
    row0 = wid * GW
    ss = [ss0, ss1]
    sd = [sd0, sd1]
    wo = [wo0, wo1]
    sem_i = [smi0, smi1]
    sem_o = [smo0, smo1]

    pltpu.sync_copy(sl_hbm, sl_v)
    pltpu.sync_copy(sr_hbm, sr_v)
    pltpu.async_copy(src_hbm.at[row0], ss[0], sem_i[0])
    pltpu.async_copy(dst_hbm.at[row0], sd[0], sem_i[0])

    def _pair(t, _):
        for b in range(2):
            g = 2 * t + b

            @pl.when(g + 1 < GW)
            def _():
                pltpu.async_copy(src_hbm.at[row0 + g + 1], ss[1 - b],
                                 sem_i[1 - b])
                pltpu.async_copy(dst_hbm.at[row0 + g + 1], sd[1 - b],
                                 sem_i[1 - b])
            pltpu.make_async_copy(src_hbm.at[row0 + g], ss[b],
                                  sem_i[b]).wait()
            pltpu.make_async_copy(dst_hbm.at[row0 + g], sd[b],
                                  sem_i[b]).wait()

            @pl.when(g >= 2)
            def _():
                pltpu.make_async_copy(wo[b], w_hbm.at[row0 + g - 2],
                                      sem_o[b]).wait()
            for j in range(CW // L):
                s16 = ss[b][pl.ds(j * L, L)]
                d16 = jnp.minimum(sd[b][pl.ds(j * L, L)], N - 1)
                e16 = (plsc.load_gather(sl_v, [s16])
                       + plsc.load_gather(sr_v, [d16]))
                e16 = jnp.where(e16 >= 0, e16, e16 * jnp.float32(0.01))
                wo[b][pl.ds(j * L, L)] = jnp.exp(e16)
            pltpu.async_copy(wo[b], w_hbm.at[row0 + g], sem_o[b])
        return 0

    lax.fori_loop(0, GW // 2, _pair, 0)
    for g in range(GW - 2, GW):
        pltpu.make_async_copy(wo[g % 2], w_hbm.at[row0 + g],
                              sem_o[g % 2]).wait()


def _agg_body(z_hbm, src_hbm, dst_hbm, w_hbm, acc_hbm, d_hbm,
              rb0, rb1, rb2, is0, is1, is2, is3, id0, id1, id2, id3,
              wv0, wv1, wv2, wv3, dzero_v, acc_sh, d_sh,
              sg0, sg1, sg2, ss0, ss1, ss2,
              smi0, smi1, smi2, smi3, smw0, smw1, smw2, smw3, sem_d):
    cid = lax.axis_index("c")
    sid = lax.axis_index("s")
    rows = [rb0, rb1, rb2]
    isv = [is0, is1, is2, is3]
    idv = [id0, id1, id2, id3]
    wv = [wv0, wv1, wv2, wv3]
    sem_g = [sg0, sg1, sg2]
    sem_s = [ss0, ss1, ss2]
    sem_i = [smi0, smi1, smi2, smi3]
    sem_w = [smw0, smw1, smw2, smw3]
    zero16 = jnp.zeros((L,), jnp.float32)

    row0 = jnp.where(cid == 0, sid * G0, NS * G0 + sid * G1)
    gw = jnp.where(cid == 0, G0, G1)

    def _zrow(r, _):
        for k in range(D // L):
            rb0[r, pl.ds(k * L, L)] = zero16
        return 0
    lax.fori_loop(0, C, _zrow, 0)

    def _zd(i, _):
        dzero_v[pl.ds(i * L, L)] = zero16
        return 0
    lax.fori_loop(0, 640 // L, _zd, 0)

    r0 = sid * RPW
    for j in range(RPW // C):
        pltpu.sync_copy(rb0, acc_sh.at[pl.ds(r0 + j * C, C)])
    rem = RPW % C
    if rem:
        pltpu.sync_copy(rb0.at[pl.ds(0, rem)],
                        acc_sh.at[pl.ds(r0 + (RPW // C) * C, rem)])
    pltpu.sync_copy(dzero_v.at[pl.ds(0, RPW)], d_sh.at[pl.ds(r0, RPW)])

    for p in range(2):
        pltpu.async_copy(src_hbm.at[row0 + p], isv[p], sem_i[p])
        pltpu.async_copy(dst_hbm.at[row0 + p], idv[p], sem_i[p])
        pltpu.async_copy(w_hbm.at[row0 + p], wv[p], sem_w[p])
    pltpu.make_async_copy(src_hbm.at[row0], isv[0], sem_i[0]).wait()
    pltpu.make_async_copy(dst_hbm.at[row0], idv[0], sem_i[0]).wait()
    pltpu.async_copy(z_hbm.at[isv[0]], rows[0], sem_g[0])
    plsc.subcore_barrier()

    def _block(t, _):
        for b in range(UNROLL):
            g = t * UNROLL + b
            br = b % NR
            brn = (b + 1) % NR
            bi = b % NI
            bin_ = (b + 1) % NI
            bi2p = (b + 2) % NI

            @pl.when(g >= 2)
            def _():
                pltpu.make_async_copy(
                    rows[brn], acc_sh.at[idv[bi2p]], sem_s[brn]).wait()
                pltpu.make_async_copy(
                    wv[bi2p], d_sh.at[idv[bi2p]], sem_d).wait()
            @pl.when(g + 2 < gw)
            def _():
                pltpu.async_copy(src_hbm.at[row0 + g + 2], isv[bi2p],
                                 sem_i[bi2p])
                pltpu.async_copy(dst_hbm.at[row0 + g + 2], idv[bi2p],
                                 sem_i[bi2p])
                pltpu.async_copy(w_hbm.at[row0 + g + 2], wv[bi2p],
                                 sem_w[bi2p])
            @pl.when(g + 1 < gw)
            def _():
                pltpu.make_async_copy(src_hbm.at[row0 + g + 1], isv[bin_],
                                      sem_i[bin_]).wait()
                pltpu.make_async_copy(dst_hbm.at[row0 + g + 1], idv[bin_],
                                      sem_i[bin_]).wait()
                pltpu.async_copy(z_hbm.at[isv[bin_]], rows[brn], sem_g[brn])
            pltpu.make_async_copy(z_hbm.at[isv[bi]], rows[br],
                                  sem_g[br]).wait()
            pltpu.make_async_copy(w_hbm.at[row0 + g], wv[bi],
                                  sem_w[bi]).wait()

            @plsc.parallel_loop(0, C, unroll=2)
            def _scale(r):
                wb = plsc.load_gather(wv[bi], [jnp.full((L,), r, jnp.int32)])
                for k in range(D // L):
                    sl_ = pl.ds(k * L, L)
                    rows[br][r, sl_] = rows[br][r, sl_] * wb

            pltpu.async_copy(rows[br], acc_sh.at[idv[bi]], sem_s[br],
                             add=True)
            pltpu.async_copy(wv[bi], d_sh.at[idv[bi]], sem_d, add=True)
        return 0

    lax.fori_loop(0, gw // UNROLL, _block, 0)
    for goff in (2, 1):
        br = (-goff) % NR
        bi = (-goff) % NI
        pltpu.make_async_copy(rows[br], acc_sh.at[idv[bi]],
                              sem_s[br]).wait()
        pltpu.make_async_copy(wv[bi], d_sh.at[idv[bi]], sem_d).wait()
    plsc.subcore_barrier()

    pltpu.sync_copy(acc_sh.at[pl.ds(r0, RPW)], acc_hbm.at[cid, pl.ds(r0, RPW)])
    pltpu.sync_copy(d_sh.at[pl.ds(r0, RPW)], dzero_v.at[pl.ds(0, RPW)])
    pltpu.sync_copy(dzero_v.at[pl.ds(0, RPW)],
                    d_hbm.at[pl.ds(cid * NPAD + r0, RPW)])


def kernel(h, edge_index, W_fc, W_attn):
    src = edge_index[0].astype(jnp.int32)
    dst = edge_index[1].astype(jnp.int32)
    pad = EPAD - E
    src_p = jnp.pad(src, (0, pad)).reshape(NS * GT, C)
    dst_p = jnp.pad(dst, (0, pad), constant_values=N).reshape(NS * GT, C)

    a_l = W_attn[0, :D]
    a_r = W_attn[0, D:]
    P = jnp.zeros((D, D), jnp.float32).at[:, 0].set(a_l).at[:, 1].set(a_r)

    blk = 1000
    z, s2 = pl.pallas_call(
        _proj_body,
        grid=(N // blk,),
        in_specs=[
            pl.BlockSpec((blk, D), lambda i: (i, 0)),
            pl.BlockSpec((D, D), lambda i: (0, 0)),
            pl.BlockSpec((D, D), lambda i: (0, 0)),
        ],
        out_specs=[
            pl.BlockSpec((blk, D), lambda i: (i, 0)),
            pl.BlockSpec((blk, D), lambda i: (i, 0)),
        ],
        out_shape=[
            jax.ShapeDtypeStruct((N, D), jnp.float32),
            jax.ShapeDtypeStruct((N, D), jnp.float32),
        ],
    )(h, W_fc, P)
    sl = s2[:, 0]
    sr = s2[:, 1]

    mesh = plsc.VectorSubcoreMesh(core_axis_name="c", subcore_axis_name="s")
    scp = pltpu.CompilerParams(needs_layout_passes=False)

    w = pl.kernel(
        _w_body,
        out_type=jax.ShapeDtypeStruct((NW * GW, CW), jnp.float32),
        mesh=mesh,
        compiler_params=scp,
        scratch_types=(
            [
                pltpu.VMEM((N,), jnp.float32),
                pltpu.VMEM((N,), jnp.float32),
            ]
            + [pltpu.VMEM((CW,), jnp.int32) for _ in range(4)]
            + [pltpu.VMEM((CW,), jnp.float32) for _ in range(2)]
            + [pltpu.SemaphoreType.DMA for _ in range(4)]
        ),
    )(sl, sr, src_p.reshape(NW * GW, CW), dst_p.reshape(NW * GW, CW))

    acc2, d2 = pl.kernel(
        _agg_body,
        out_type=[
            jax.ShapeDtypeStruct((NC, NPAD, D), jnp.float32),
            jax.ShapeDtypeStruct((NC * NPAD,), jnp.float32),
        ],
        mesh=mesh,
        compiler_params=scp,
        scratch_types=(
            [pltpu.VMEM((C, D), jnp.float32) for _ in range(NR)]
            + [pltpu.VMEM((C,), jnp.int32) for _ in range(2 * NI)]
            + [pltpu.VMEM((C,), jnp.float32) for _ in range(NI)]
            + [
                pltpu.VMEM((640,), jnp.float32),
                pltpu.MemorySpace.VMEM_SHARED((NPAD, D), jnp.float32),
                pltpu.MemorySpace.VMEM_SHARED((NPAD,), jnp.float32),
            ]
            + [pltpu.SemaphoreType.DMA for _ in range(15)]
        ),
    )(z, src_p, dst_p, w.reshape(NS * GT, C))

    out = pl.pallas_call(
        _combine_body,
        grid=(N // blk,),
        in_specs=[
            pl.BlockSpec((NC, blk, D), lambda i: (0, i, 0)),
            pl.BlockSpec((NC, blk, 1), lambda i: (0, i, 0)),
        ],
        out_specs=pl.BlockSpec((blk, D), lambda i: (i, 0)),
        out_shape=jax.ShapeDtypeStruct((N, D), jnp.float32),
    )(acc2, d2.reshape(NC, NPAD, 1))
    return out


# speedup vs baseline: 1.0984x; 1.0292x over previous
import jax
import jax.numpy as jnp
from jax import lax
from jax.experimental import pallas as pl
from jax.experimental.pallas import tpu as pltpu
from jax.experimental.pallas import tpu_sc as plsc

N = 10000
D = 128
E = 320000

NC = 2
NS = 16
NW = NC * NS
L = 16

C = 120
NR = 3
NI = 4
UNROLL = 12
GT = 168
G0 = 132
G1 = GT - G0
EPAD = NS * GT * C
CW = 1008
GW = EPAD // (NW * CW)
NPAD = 10112
RPW = NPAD // NS


def _proj_body(h_ref, wfc_ref, p_ref, z_ref, s2_ref):
    z = lax.dot_general(h_ref[...], wfc_ref[...], (((1,), (1,)), ((), ())),
                        preferred_element_type=jnp.float32)
    z_ref[...] = z
    s2_ref[...] = lax.dot_general(z, p_ref[...], (((1,), (0,)), ((), ())),
                                  preferred_element_type=jnp.float32)


def _combine_body(acc_ref, d_ref, out_ref):
    a = acc_ref[0] + acc_ref[1]
    dsum = d_ref[0] + d_ref[1]
    safe = jnp.where(dsum > 0, dsum, 1.0)
    out_ref[...] = jnp.where(dsum > 0, a / safe, 0.0)


def _w_body(sl_hbm, sr_hbm, src_hbm, dst_hbm, w_hbm,
            sl_v, sr_v, ss0, ss1, sd0, sd1, wo0, wo1,
            smi0, smi1, smo0, smo1):
    cid = lax.axis_index("c")
    sid = lax.axis_index("s")
    wid = sid * NC + cid
    row0 = wid * GW
    ss = [ss0, ss1]
    sd = [sd0, sd1]
    wo = [wo0, wo1]
    sem_i = [smi0, smi1]
    sem_o = [smo0, smo1]

    pltpu.sync_copy(sl_hbm, sl_v)
    pltpu.sync_copy(sr_hbm, sr_v)
    pltpu.async_copy(src_hbm.at[row0], ss[0], sem_i[0])
    pltpu.async_copy(dst_hbm.at[row0], sd[0], sem_i[0])

    def _pair(t, _):
        for b in range(2):
            g = 2 * t + b

            @pl.when(g + 1 < GW)
            def _():
                pltpu.async_copy(src_hbm.at[row0 + g + 1], ss[1 - b],
                                 sem_i[1 - b])
                pltpu.async_copy(dst_hbm.at[row0 + g + 1], sd[1 - b],
                                 sem_i[1 - b])
            pltpu.make_async_copy(src_hbm.at[row0 + g], ss[b],
                                  sem_i[b]).wait()
            pltpu.make_async_copy(dst_hbm.at[row0 + g], sd[b],
                                  sem_i[b]).wait()

            @pl.when(g >= 2)
            def _():
                pltpu.make_async_copy(wo[b], w_hbm.at[row0 + g - 2],
                                      sem_o[b]).wait()
            for j in range(CW // L):
                s16 = ss[b][pl.ds(j * L, L)]
                d16 = jnp.minimum(sd[b][pl.ds(j * L, L)], N - 1)
                e16 = (plsc.load_gather(sl_v, [s16])
                       + plsc.load_gather(sr_v, [d16]))
                e16 = jnp.where(e16 >= 0, e16, e16 * jnp.float32(0.01))
                wo[b][pl.ds(j * L, L)] = jnp.exp(e16)
            pltpu.async_copy(wo[b], w_hbm.at[row0 + g], sem_o[b])
        return 0

    lax.fori_loop(0, GW // 2, _pair, 0)
    for g in range(GW - 2, GW):
        pltpu.make_async_copy(wo[g % 2], w_hbm.at[row0 + g],
                              sem_o[g % 2]).wait()


def _agg_body(z_hbm, src_hbm, dst_hbm, w_hbm, acc_hbm, d_hbm,
              rb0, rb1, rb2, is0, is1, is2, is3, id0, id1, id2, id3,
              wv0, wv1, wv2, wv3, dzero_v, acc_sh, d_sh,
              sg0, sg1, sg2, ss0, ss1, ss2,
              smi0, smi1, smi2, smi3, smw0, smw1, smw2, smw3, sem_d):
    cid = lax.axis_index("c")
    sid = lax.axis_index("s")
    rows = [rb0, rb1, rb2]
    isv = [is0, is1, is2, is3]
    idv = [id0, id1, id2, id3]
    wv = [wv0, wv1, wv2, wv3]
    sem_g = [sg0, sg1, sg2]
    sem_s = [ss0, ss1, ss2]
    sem_i = [smi0, smi1, smi2, smi3]
    sem_w = [smw0, smw1, smw2, smw3]
    zero16 = jnp.zeros((L,), jnp.float32)

    row0 = jnp.where(cid == 0, sid * G0, NS * G0 + sid * G1)
    gw = jnp.where(cid == 0, G0, G1)

    def _zrow(r, _):
        for k in range(D // L):
            rb0[r, pl.ds(k * L, L)] = zero16
        return 0
    lax.fori_loop(0, C, _zrow, 0)

    def _zd(i, _):
        dzero_v[pl.ds(i * L, L)] = zero16
        return 0
    lax.fori_loop(0, 640 // L, _zd, 0)

    r0 = sid * RPW
    for j in range(RPW // C):
        pltpu.sync_copy(rb0, acc_sh.at[pl.ds(r0 + j * C, C)])
    rem = RPW % C
    if rem:
        pltpu.sync_copy(rb0.at[pl.ds(0, rem)],
                        acc_sh.at[pl.ds(r0 + (RPW // C) * C, rem)])
    pltpu.sync_copy(dzero_v.at[pl.ds(0, RPW)], d_sh.at[pl.ds(r0, RPW)])

    for p in range(2):
        pltpu.async_copy(src_hbm.at[row0 + p], isv[p], sem_i[p])
        pltpu.async_copy(dst_hbm.at[row0 + p], idv[p], sem_i[p])
        pltpu.async_copy(w_hbm.at[row0 + p], wv[p], sem_w[p])
    pltpu.make_async_copy(src_hbm.at[row0], isv[0], sem_i[0]).wait()
    pltpu.make_async_copy(dst_hbm.at[row0], idv[0], sem_i[0]).wait()
    pltpu.async_copy(z_hbm.at[isv[0]], rows[0], sem_g[0])
    plsc.subcore_barrier()

    def _block(t, _):
        for b in range(UNROLL):
            g = t * UNROLL + b
            br = b % NR
            brn = (b + 1) % NR
            bi = b % NI
            bin_ = (b + 1) % NI
            bi2p = (b + 2) % NI

            @pl.when(g >= 2)
            def _():
                pltpu.make_async_copy(
                    rows[brn], acc_sh.at[idv[bi2p]], sem_s[brn]).wait()
                pltpu.make_async_copy(
                    wv[bi2p], d_sh.at[idv[bi2p]], sem_d).wait()
            @pl.when(g + 2 < gw)
            def _():
                pltpu.async_copy(src_hbm.at[row0 + g + 2], isv[bi2p],
                                 sem_i[bi2p])
                pltpu.async_copy(dst_hbm.at[row0 + g + 2], idv[bi2p],
                                 sem_i[bi2p])
                pltpu.async_copy(w_hbm.at[row0 + g + 2], wv[bi2p],
                                 sem_w[bi2p])
            @pl.when(g + 1 < gw)
            def _():
                pltpu.make_async_copy(src_hbm.at[row0 + g + 1], isv[bin_],
                                      sem_i[bin_]).wait()
                pltpu.make_async_copy(dst_hbm.at[row0 + g + 1], idv[bin_],
                                      sem_i[bin_]).wait()
                pltpu.async_copy(z_hbm.at[isv[bin_]], rows[brn], sem_g[brn])
            pltpu.make_async_copy(z_hbm.at[isv[bi]], rows[br],
                                  sem_g[br]).wait()
            pltpu.make_async_copy(w_hbm.at[row0 + g], wv[bi],
                                  sem_w[bi]).wait()

            @plsc.parallel_loop(0, C, unroll=2)
            def _scale(r):
                wb = plsc.load_gather(wv[bi], [jnp.full((L,), r, jnp.int32)])
                for k in range(D // L):
                    sl_ = pl.ds(k * L, L)
                    rows[br][r, sl_] = rows[br][r, sl_] * wb

            pltpu.async_copy(rows[br], acc_sh.at[idv[bi]], sem_s[br],
                             add=True)
            pltpu.async_copy(wv[bi], d_sh.at[idv[bi]], sem_d, add=True)
        return 0

    lax.fori_loop(0, gw // UNROLL, _block, 0)
    for goff in (2, 1):
        br = (-goff) % NR
        bi = (-goff) % NI
        pltpu.make_async_copy(rows[br], acc_sh.at[idv[bi]],
                              sem_s[br]).wait()
        pltpu.make_async_copy(wv[bi], d_sh.at[idv[bi]], sem_d).wait()
    plsc.subcore_barrier()

    pltpu.sync_copy(acc_sh.at[pl.ds(r0, RPW)], acc_hbm.at[cid, pl.ds(r0, RPW)])
    pltpu.sync_copy(d_sh.at[pl.ds(r0, RPW)], dzero_v.at[pl.ds(0, RPW)])
    pltpu.sync_copy(dzero_v.at[pl.ds(0, RPW)],
                    d_hbm.at[pl.ds(cid * NPAD + r0, RPW)])


def kernel(h, edge_index, W_fc, W_attn):
    src = edge_index[0].astype(jnp.int32)
    dst = edge_index[1].astype(jnp.int32)
    pad = EPAD - E
    src_p = jnp.pad(src, (0, pad)).reshape(NS * GT, C)
    dst_p = jnp.pad(dst, (0, pad), constant_values=N).reshape(NS * GT, C)

    a_l = W_attn[0, :D]
    a_r = W_attn[0, D:]
    P = jnp.zeros((D, D), jnp.float32).at[:, 0].set(a_l).at[:, 1].set(a_r)

    blk = 1000
    z, s2 = pl.pallas_call(
        _proj_body,
        grid=(N // blk,),
        in_specs=[
            pl.BlockSpec((blk, D), lambda i: (i, 0)),
            pl.BlockSpec((D, D), lambda i: (0, 0)),
            pl.BlockSpec((D, D), lambda i: (0, 0)),
        ],
        out_specs=[
            pl.BlockSpec((blk, D), lambda i: (i, 0)),
            pl.BlockSpec((blk, D), lambda i: (i, 0)),
        ],
        out_shape=[
            jax.ShapeDtypeStruct((N, D), jnp.float32),
            jax.ShapeDtypeStruct((N, D), jnp.float32),
        ],
    )(h, W_fc, P)
    sl = s2[:, 0]
    sr = s2[:, 1]

    mesh = plsc.VectorSubcoreMesh(core_axis_name="c", subcore_axis_name="s")
    scp = pltpu.CompilerParams(needs_layout_passes=False)

    w = pl.kernel(
        _w_body,
        out_type=jax.ShapeDtypeStruct((NW * GW, CW), jnp.float32),
        mesh=mesh,
        compiler_params=scp,
        scratch_types=(
            [
                pltpu.VMEM((N,), jnp.float32),
                pltpu.VMEM((N,), jnp.float32),
            ]
            + [pltpu.VMEM((CW,), jnp.int32) for _ in range(4)]
            + [pltpu.VMEM((CW,), jnp.float32) for _ in range(2)]
            + [pltpu.SemaphoreType.DMA for _ in range(4)]
        ),
    )(sl, sr, src_p.reshape(NW * GW, CW), dst_p.reshape(NW * GW, CW))

    acc2, d2 = pl.kernel(
        _agg_body,
        out_type=[
            jax.ShapeDtypeStruct((NC, NPAD, D), jnp.float32),
            jax.ShapeDtypeStruct((NC * NPAD,), jnp.float32),
        ],
        mesh=mesh,
        compiler_params=scp,
        scratch_types=(
            [pltpu.VMEM((C, D), jnp.float32) for _ in range(NR)]
            + [pltpu.VMEM((C,), jnp.int32) for _ in range(2 * NI)]
            + [pltpu.VMEM((C,), jnp.float32) for _ in range(NI)]
            + [
                pltpu.VMEM((640,), jnp.float32),
                pltpu.MemorySpace.VMEM_SHARED((NPAD, D), jnp.float32),
                pltpu.MemorySpace.VMEM_SHARED((NPAD,), jnp.float32),
            ]
            + [pltpu.SemaphoreType.DMA for _ in range(15)]
        ),
    )(z, src_p, dst_p, w.reshape(NS * GT, C))

    out = pl.pallas_call(
        _combine_body,
        grid=(N // blk,),
        in_specs=[
            pl.BlockSpec((NC, blk, D), lambda i: (0, i, 0)),
            pl.BlockSpec((NC, blk, 1), lambda i: (0, i, 0)),
        ],
        out_specs=pl.BlockSpec((blk, D), lambda i: (i, 0)),
        out_shape=jax.ShapeDtypeStruct((N, D), jnp.float32),
    )(acc2, d2.reshape(NC, NPAD, 1))
    return out
